# baseline (speedup 1.0000x reference)
import functools

import jax
import jax.numpy as jnp
from jax import lax
from jax.experimental import pallas as pl
from jax.experimental.pallas import tpu as pltpu
from jax.experimental.pallas import tpu_sc as plsc

N = 10000
K = 16
IN_DIM = 256
OUT_DIM = 256
EPS = 1e-5

NA = 4000
BN = 1000
NB_A = NA // BN
NB_B = (N - NA) // BN
NB_ALL = N // BN

NC = 2
NS = 16
NW = NC * NS
BASE = 128
TAIL = NA - (NW - 1) * BASE
C = 8
CK = C * K
NCHUNK = BASE // C


def _sc_agg(neigh2d):
    mesh = plsc.VectorSubcoreMesh(core_axis_name="c", subcore_axis_name="s")

    @functools.partial(
        pl.kernel,
        out_type=jax.ShapeDtypeStruct((NA, IN_DIM), jnp.float32),
        mesh=mesh,
        scratch_types=[
            pltpu.VMEM((CK, IN_DIM), jnp.float32),
            pltpu.VMEM((CK, IN_DIM), jnp.float32),
            pltpu.VMEM((BASE, IN_DIM), jnp.float32),
            pltpu.SemaphoreType.DMA,
            pltpu.SemaphoreType.DMA,
        ],
    )
    def sck(neigh_hbm, out_hbm, buf_a, buf_b, acc_v, sem_a, sem_b):
        cid = lax.axis_index("c")
        sid = lax.axis_index("s")
        wid = sid * NC + cid
        node0 = wid * BASE
        my_nchunk = jnp.where(wid == NW - 1, TAIL // C, NCHUNK)

        def start(t, buf, sem):
            pltpu.async_copy(
                neigh_hbm.at[pl.ds((node0 + t * C) * K, CK)], buf, sem)

        def wait(buf, sem):
            pltpu.make_async_copy(neigh_hbm.at[pl.ds(0, CK)], buf, sem).wait()

        def reduce_chunk(t, buf):
            for n in range(C):
                row0 = n * K
                for c16 in range(IN_DIM // 16):
                    sl = pl.ds(c16 * 16, 16)
                    vals = [buf[row0 + k, sl] for k in range(K)]
                    while len(vals) > 1:
                        vals = [a + b for a, b in zip(vals[::2], vals[1::2])]
                    acc_v[t * C + n, sl] = vals[0]

        start(0, buf_a, sem_a)

        @pl.loop(0, (NCHUNK + 1) // 2)
        def _pair(p):
            t0 = 2 * p
            t1 = t0 + 1

            @pl.when(t1 < my_nchunk)
            def _():
                start(t1, buf_b, sem_b)

            @pl.when(t0 < my_nchunk)
            def _():
                wait(buf_a, sem_a)
                reduce_chunk(t0, buf_a)

            @pl.when(t0 + 2 < my_nchunk)
            def _():
                start(t0 + 2, buf_a, sem_a)

            @pl.when(t1 < my_nchunk)
            def _():
                wait(buf_b, sem_b)
                reduce_chunk(t1, buf_b)

        @pl.when(wid < NW - 1)
        def _full_out():
            pltpu.sync_copy(acc_v.at[pl.ds(0, BASE)],
                            out_hbm.at[pl.ds(node0, BASE)])

        @pl.when(wid == NW - 1)
        def _tail_out():
            pltpu.sync_copy(acc_v.at[pl.ds(0, TAIL)],
                            out_hbm.at[pl.ds(node0, TAIL)])

    return sck(neigh2d)


def _tc1_body(self_ref, neigh_ref, ws_ref, wn_ref, b_ref, y_ref, stats_ref,
              acc_ref):
    i = pl.program_id(0)

    @pl.when(i == 0)
    def _init():
        acc_ref[...] = jnp.zeros_like(acc_ref)

    agg = jnp.mean(neigh_ref[...], axis=1)
    y = (
        jnp.dot(self_ref[...], ws_ref[...], preferred_element_type=jnp.float32)
        + jnp.dot(agg, wn_ref[...], preferred_element_type=jnp.float32)
        + b_ref[...]
    )
    y_ref[...] = y
    acc_ref[0:1, :] += jnp.sum(y, axis=0, keepdims=True)
    acc_ref[1:2, :] += jnp.sum(y * y, axis=0, keepdims=True)

    @pl.when(i == NB_B - 1)
    def _emit():
        stats_ref[...] = acc_ref[...]


def _tc2_body(self_ref, agg_ref, yb_ref, stats_ref, ws_ref, wns_ref, b_ref,
              gamma_ref, beta_ref, out_ref, y_scr, acc_ref):
    i = pl.program_id(0)

    @pl.when(i == 0)
    def _init():
        acc_ref[...] = stats_ref[...]

    @pl.when(i < NB_A)
    def _compute():
        y = (
            jnp.dot(self_ref[...], ws_ref[...], preferred_element_type=jnp.float32)
            + jnp.dot(agg_ref[...], wns_ref[...], preferred_element_type=jnp.float32)
            + b_ref[...]
        )
        y_scr[pl.ds(i * BN, BN), :] = y
        acc_ref[0:1, :] += jnp.sum(y, axis=0, keepdims=True)
        acc_ref[1:2, :] += jnp.sum(y * y, axis=0, keepdims=True)

    @pl.when(i >= NB_A)
    def _normalize():
        j = i - NB_A
        mean = acc_ref[0:1, :] / N
        var = acc_ref[1:2, :] / N - mean * mean
        scale = gamma_ref[...] * jax.lax.rsqrt(var + EPS)
        shift = beta_ref[...] - mean * scale

        @pl.when(j < NB_A)
        def _from_scratch():
            y = y_scr[pl.ds(j * BN, BN), :]
            out_ref[...] = jnp.maximum(y * scale + shift, 0.0)

        @pl.when(j >= NB_A)
        def _from_yb():
            out_ref[...] = jnp.maximum(yb_ref[...] * scale + shift, 0.0)


@jax.jit
def kernel(self_feat, neighbor_feat, W, b, gamma, beta):
    ws = W[:, :IN_DIM].T
    wn = W[:, IN_DIM:].T
    wns = wn * (1.0 / K)
    b2 = b.reshape(1, OUT_DIM)
    gamma2 = gamma.reshape(1, OUT_DIM)
    beta2 = beta.reshape(1, OUT_DIM)

    neigh2d = neighbor_feat.reshape(N * K, IN_DIM)
    agg_a = _sc_agg(neigh2d)

    y_b, stats = pl.pallas_call(
        _tc1_body,
        grid=(NB_B,),
        in_specs=[
            pl.BlockSpec((BN, IN_DIM), lambda i: (NB_A + i, 0)),
            pl.BlockSpec((BN, K, IN_DIM), lambda i: (NB_A + i, 0, 0)),
            pl.BlockSpec((IN_DIM, OUT_DIM), lambda i: (0, 0)),
            pl.BlockSpec((IN_DIM, OUT_DIM), lambda i: (0, 0)),
            pl.BlockSpec((1, OUT_DIM), lambda i: (0, 0)),
        ],
        out_specs=[
            pl.BlockSpec((BN, OUT_DIM), lambda i: (i, 0)),
            pl.BlockSpec((2, OUT_DIM), lambda i: (0, 0)),
        ],
        out_shape=[
            jax.ShapeDtypeStruct((N - NA, OUT_DIM), jnp.float32),
            jax.ShapeDtypeStruct((2, OUT_DIM), jnp.float32),
        ],
        scratch_shapes=[pltpu.VMEM((2, OUT_DIM), jnp.float32)],
    )(self_feat, neighbor_feat, ws, wn, b2)

    out = pl.pallas_call(
        _tc2_body,
        grid=(NB_A + NB_ALL,),
        in_specs=[
            pl.BlockSpec((BN, IN_DIM), lambda i: (jnp.minimum(i, NB_A - 1), 0)),
            pl.BlockSpec((BN, IN_DIM), lambda i: (jnp.minimum(i, NB_A - 1), 0)),
            pl.BlockSpec(
                (BN, OUT_DIM),
                lambda i: (jnp.clip(i - 2 * NB_A, 0, NB_B - 1), 0),
            ),
            pl.BlockSpec((2, OUT_DIM), lambda i: (0, 0)),
            pl.BlockSpec((IN_DIM, OUT_DIM), lambda i: (0, 0)),
            pl.BlockSpec((IN_DIM, OUT_DIM), lambda i: (0, 0)),
            pl.BlockSpec((1, OUT_DIM), lambda i: (0, 0)),
            pl.BlockSpec((1, OUT_DIM), lambda i: (0, 0)),
            pl.BlockSpec((1, OUT_DIM), lambda i: (0, 0)),
        ],
        out_specs=pl.BlockSpec((BN, OUT_DIM), lambda i: (jnp.maximum(i - NB_A, 0), 0)),
        out_shape=jax.ShapeDtypeStruct((N, OUT_DIM), jnp.float32),
        scratch_shapes=[
            pltpu.VMEM((NA, OUT_DIM), jnp.float32),
            pltpu.VMEM((2, OUT_DIM), jnp.float32),
        ],
    )(self_feat, agg_a, y_b, stats, ws, wns, b2, gamma2, beta2)
    return out


# speedup vs baseline: 1.6854x; 1.6854x over previous
import functools

import jax
import jax.numpy as jnp
from jax import lax
from jax.experimental import pallas as pl
from jax.experimental.pallas import tpu as pltpu
from jax.experimental.pallas import tpu_sc as plsc

N = 10000
K = 16
IN_DIM = 256
OUT_DIM = 256
EPS = 1e-5

NA = 4000
BN = 1000
NB_A = NA // BN
NB_B = (N - NA) // BN
NB_ALL = N // BN

NC = 2
NS = 16
NW = NC * NS
BASE = 128
TAIL = NA - (NW - 1) * BASE
C = 4
CK = C * K
NCHUNK = BASE // C


def _sc_agg(neigh2d):
    mesh = plsc.VectorSubcoreMesh(core_axis_name="c", subcore_axis_name="s")

    @functools.partial(
        pl.kernel,
        out_type=jax.ShapeDtypeStruct((NA, IN_DIM), jnp.float32),
        mesh=mesh,
        cost_estimate=pl.CostEstimate(
            flops=2 * NA * K * IN_DIM,
            bytes_accessed=NA * K * IN_DIM * 4 + NA * IN_DIM * 4,
            transcendentals=0,
        ),
        scratch_types=[
            pltpu.VMEM((CK, IN_DIM), jnp.float32),
            pltpu.VMEM((CK, IN_DIM), jnp.float32),
            pltpu.VMEM((BASE, IN_DIM), jnp.float32),
            pltpu.SemaphoreType.DMA,
            pltpu.SemaphoreType.DMA,
        ],
    )
    def sck(neigh_hbm, out_hbm, buf_a, buf_b, acc_v, sem_a, sem_b):
        cid = lax.axis_index("c")
        sid = lax.axis_index("s")
        wid = sid * NC + cid
        node0 = wid * BASE
        my_nchunk = jnp.where(wid == NW - 1, TAIL // C, NCHUNK)

        def start(t, buf, sem):
            pltpu.async_copy(
                neigh_hbm.at[pl.ds((node0 + t * C) * K, CK)], buf, sem)

        def wait(buf, sem):
            pltpu.make_async_copy(neigh_hbm.at[pl.ds(0, CK)], buf, sem).wait()

        def reduce_chunk(t, buf):
            @pl.loop(0, C)
            def _node(n):
                row0 = n * K
                for c16 in range(IN_DIM // 16):
                    sl = pl.ds(c16 * 16, 16)
                    vals = [buf[row0 + k, sl] for k in range(K)]
                    while len(vals) > 1:
                        vals = [a + b for a, b in zip(vals[::2], vals[1::2])]
                    acc_v[t * C + n, sl] = vals[0]

        start(0, buf_a, sem_a)

        @pl.loop(0, (NCHUNK + 1) // 2)
        def _pair(p):
            t0 = 2 * p
            t1 = t0 + 1

            @pl.when(t1 < my_nchunk)
            def _():
                start(t1, buf_b, sem_b)

            @pl.when(t0 < my_nchunk)
            def _():
                wait(buf_a, sem_a)
                reduce_chunk(t0, buf_a)

            @pl.when(t0 + 2 < my_nchunk)
            def _():
                start(t0 + 2, buf_a, sem_a)

            @pl.when(t1 < my_nchunk)
            def _():
                wait(buf_b, sem_b)
                reduce_chunk(t1, buf_b)

        @pl.when(wid < NW - 1)
        def _full_out():
            pltpu.sync_copy(acc_v.at[pl.ds(0, BASE)],
                            out_hbm.at[pl.ds(node0, BASE)])

        @pl.when(wid == NW - 1)
        def _tail_out():
            pltpu.sync_copy(acc_v.at[pl.ds(0, TAIL)],
                            out_hbm.at[pl.ds(node0, TAIL)])

    return sck(neigh2d)


def _tc1_body(self_ref, neigh_ref, ws_ref, wn_ref, b_ref, y_ref, stats_ref,
              acc_ref):
    i = pl.program_id(0)

    @pl.when(i == 0)
    def _init():
        acc_ref[...] = jnp.zeros_like(acc_ref)

    agg = jnp.mean(neigh_ref[...], axis=1)
    y = (
        jnp.dot(self_ref[...], ws_ref[...], preferred_element_type=jnp.float32)
        + jnp.dot(agg, wn_ref[...], preferred_element_type=jnp.float32)
        + b_ref[...]
    )
    y_ref[...] = y
    acc_ref[0:1, :] += jnp.sum(y, axis=0, keepdims=True)
    acc_ref[1:2, :] += jnp.sum(y * y, axis=0, keepdims=True)

    @pl.when(i == NB_B - 1)
    def _emit():
        stats_ref[...] = acc_ref[...]


def _tc2_body(self_ref, agg_ref, yb_ref, stats_ref, ws_ref, wns_ref, b_ref,
              gamma_ref, beta_ref, out_ref, y_scr, acc_ref):
    i = pl.program_id(0)

    @pl.when(i == 0)
    def _init():
        acc_ref[...] = stats_ref[...]

    @pl.when(i < NB_A)
    def _compute():
        y = (
            jnp.dot(self_ref[...], ws_ref[...], preferred_element_type=jnp.float32)
            + jnp.dot(agg_ref[...], wns_ref[...], preferred_element_type=jnp.float32)
            + b_ref[...]
        )
        y_scr[pl.ds(i * BN, BN), :] = y
        acc_ref[0:1, :] += jnp.sum(y, axis=0, keepdims=True)
        acc_ref[1:2, :] += jnp.sum(y * y, axis=0, keepdims=True)

    @pl.when(i >= NB_A)
    def _normalize():
        j = i - NB_A
        mean = acc_ref[0:1, :] / N
        var = acc_ref[1:2, :] / N - mean * mean
        scale = gamma_ref[...] * jax.lax.rsqrt(var + EPS)
        shift = beta_ref[...] - mean * scale

        @pl.when(j < NB_A)
        def _from_scratch():
            y = y_scr[pl.ds(j * BN, BN), :]
            out_ref[...] = jnp.maximum(y * scale + shift, 0.0)

        @pl.when(j >= NB_A)
        def _from_yb():
            out_ref[...] = jnp.maximum(yb_ref[...] * scale + shift, 0.0)


@jax.jit
def kernel(self_feat, neighbor_feat, W, b, gamma, beta):
    ws = W[:, :IN_DIM].T
    wn = W[:, IN_DIM:].T
    wns = wn * (1.0 / K)
    b2 = b.reshape(1, OUT_DIM)
    gamma2 = gamma.reshape(1, OUT_DIM)
    beta2 = beta.reshape(1, OUT_DIM)

    neigh2d = neighbor_feat.reshape(N * K, IN_DIM)
    agg_a = _sc_agg(neigh2d)

    y_b, stats = pl.pallas_call(
        _tc1_body,
        grid=(NB_B,),
        in_specs=[
            pl.BlockSpec((BN, IN_DIM), lambda i: (NB_A + i, 0)),
            pl.BlockSpec((BN, K, IN_DIM), lambda i: (NB_A + i, 0, 0)),
            pl.BlockSpec((IN_DIM, OUT_DIM), lambda i: (0, 0)),
            pl.BlockSpec((IN_DIM, OUT_DIM), lambda i: (0, 0)),
            pl.BlockSpec((1, OUT_DIM), lambda i: (0, 0)),
        ],
        out_specs=[
            pl.BlockSpec((BN, OUT_DIM), lambda i: (i, 0)),
            pl.BlockSpec((2, OUT_DIM), lambda i: (0, 0)),
        ],
        out_shape=[
            jax.ShapeDtypeStruct((N - NA, OUT_DIM), jnp.float32),
            jax.ShapeDtypeStruct((2, OUT_DIM), jnp.float32),
        ],
        scratch_shapes=[pltpu.VMEM((2, OUT_DIM), jnp.float32)],
    )(self_feat, neighbor_feat, ws, wn, b2)

    out = pl.pallas_call(
        _tc2_body,
        grid=(NB_A + NB_ALL,),
        in_specs=[
            pl.BlockSpec((BN, IN_DIM), lambda i: (jnp.minimum(i, NB_A - 1), 0)),
            pl.BlockSpec((BN, IN_DIM), lambda i: (jnp.minimum(i, NB_A - 1), 0)),
            pl.BlockSpec(
                (BN, OUT_DIM),
                lambda i: (jnp.clip(i - 2 * NB_A, 0, NB_B - 1), 0),
            ),
            pl.BlockSpec((2, OUT_DIM), lambda i: (0, 0)),
            pl.BlockSpec((IN_DIM, OUT_DIM), lambda i: (0, 0)),
            pl.BlockSpec((IN_DIM, OUT_DIM), lambda i: (0, 0)),
            pl.BlockSpec((1, OUT_DIM), lambda i: (0, 0)),
            pl.BlockSpec((1, OUT_DIM), lambda i: (0, 0)),
            pl.BlockSpec((1, OUT_DIM), lambda i: (0, 0)),
        ],
        out_specs=pl.BlockSpec((BN, OUT_DIM), lambda i: (jnp.maximum(i - NB_A, 0), 0)),
        out_shape=jax.ShapeDtypeStruct((N, OUT_DIM), jnp.float32),
        scratch_shapes=[
            pltpu.VMEM((NA, OUT_DIM), jnp.float32),
            pltpu.VMEM((2, OUT_DIM), jnp.float32),
        ],
    )(self_feat, agg_a, y_b, stats, ws, wns, b2, gamma2, beta2)
    return out


# speedup vs baseline: 2.2848x; 1.3557x over previous
import functools

import jax
import jax.numpy as jnp
from jax.experimental import pallas as pl
from jax.experimental.pallas import tpu as pltpu

N = 10000
K = 16
IN_DIM = 256
OUT_DIM = 256
BN = 1000
NB = N // BN
EPS = 1e-5


def _body(self_ref, neigh1_ref, neigh2_ref, ws_ref, wn_ref, b_ref, gamma_ref,
          beta_ref, out_ref, y_ref, acc_ref):
    i = pl.program_id(0)

    @pl.when(i == 0)
    def _init():
        acc_ref[...] = jnp.zeros_like(acc_ref)

    @pl.when(i < NB)
    def _compute():
        agg = jnp.sum(neigh1_ref[...], axis=1) + jnp.sum(neigh2_ref[...], axis=1)
        y = (
            jnp.dot(self_ref[...], ws_ref[...], preferred_element_type=jnp.float32)
            + jnp.dot(agg, wn_ref[...], preferred_element_type=jnp.float32)
            + b_ref[...]
        )
        y_ref[pl.ds(i * BN, BN), :] = y
        acc_ref[0:1, :] += jnp.sum(y, axis=0, keepdims=True)
        acc_ref[1:2, :] += jnp.sum(y * y, axis=0, keepdims=True)

    @pl.when(i >= NB)
    def _normalize():
        j = i - NB
        mean = acc_ref[0:1, :] / N
        var = acc_ref[1:2, :] / N - mean * mean
        scale = gamma_ref[...] * jax.lax.rsqrt(var + EPS)
        shift = beta_ref[...] - mean * scale
        y = y_ref[pl.ds(j * BN, BN), :]
        out_ref[...] = jnp.maximum(y * scale + shift, 0.0)


@jax.jit
def kernel(self_feat, neighbor_feat, W, b, gamma, beta):
    ws = W[:, :IN_DIM].T
    wn = W[:, IN_DIM:].T * (1.0 / K)
    b2 = b.reshape(1, OUT_DIM)
    gamma2 = gamma.reshape(1, OUT_DIM)
    beta2 = beta.reshape(1, OUT_DIM)

    grid = (2 * NB,)
    out = pl.pallas_call(
        _body,
        grid=grid,
        in_specs=[
            pl.BlockSpec((BN, IN_DIM), lambda i: (jnp.minimum(i, NB - 1), 0)),
            pl.BlockSpec((BN, K // 2, IN_DIM),
                         lambda i: (jnp.minimum(i, NB - 1), 0, 0)),
            pl.BlockSpec((BN, K // 2, IN_DIM),
                         lambda i: (jnp.minimum(i, NB - 1), 1, 0)),
            pl.BlockSpec((IN_DIM, OUT_DIM), lambda i: (0, 0)),
            pl.BlockSpec((IN_DIM, OUT_DIM), lambda i: (0, 0)),
            pl.BlockSpec((1, OUT_DIM), lambda i: (0, 0)),
            pl.BlockSpec((1, OUT_DIM), lambda i: (0, 0)),
            pl.BlockSpec((1, OUT_DIM), lambda i: (0, 0)),
        ],
        out_specs=pl.BlockSpec((BN, OUT_DIM), lambda i: (jnp.maximum(i - NB, 0), 0)),
        out_shape=jax.ShapeDtypeStruct((N, OUT_DIM), jnp.float32),
        scratch_shapes=[
            pltpu.VMEM((N, OUT_DIM), jnp.float32),
            pltpu.VMEM((2, OUT_DIM), jnp.float32),
        ],
        compiler_params=pltpu.CompilerParams(
            vmem_limit_bytes=100 * 1024 * 1024,
        ),
    )(self_feat, neighbor_feat, neighbor_feat, ws, wn, b2, gamma2, beta2)
    return out


# speedup vs baseline: 2.4405x; 1.0681x over previous
import functools

import jax
import jax.numpy as jnp
from jax.experimental import pallas as pl
from jax.experimental.pallas import tpu as pltpu

N = 10000
K = 16
IN_DIM = 256
OUT_DIM = 256
BN = 1000
NB = N // BN
BNO = 2000
NBO = N // BNO
EPS = 1e-5


def _body(self_ref, neigh_ref, ws_ref, wn_ref, b_ref, gamma_ref,
          beta_ref, out_ref, y_ref, acc_ref):
    i = pl.program_id(0)

    @pl.when(i == 0)
    def _init():
        acc_ref[...] = jnp.zeros_like(acc_ref)

    @pl.when(i < NB)
    def _compute():
        agg = jnp.sum(neigh_ref[...], axis=1)
        y = (
            jnp.dot(self_ref[...], ws_ref[...], preferred_element_type=jnp.float32)
            + jnp.dot(agg, wn_ref[...], preferred_element_type=jnp.float32)
            + b_ref[...]
        )
        y_ref[pl.ds(i * BN, BN), :] = y
        acc_ref[0:1, :] += jnp.sum(y, axis=0, keepdims=True)
        acc_ref[1:2, :] += jnp.sum(y * y, axis=0, keepdims=True)

    @pl.when(i >= NB)
    def _normalize():
        j = i - NB
        mean = acc_ref[0:1, :] / N
        var = acc_ref[1:2, :] / N - mean * mean
        scale = gamma_ref[...] * jax.lax.rsqrt(var + EPS)
        shift = beta_ref[...] - mean * scale
        y = y_ref[pl.ds(j * BNO, BNO), :]
        out_ref[...] = jnp.maximum(y * scale + shift, 0.0)


@jax.jit
def kernel(self_feat, neighbor_feat, W, b, gamma, beta):
    ws = W[:, :IN_DIM].T
    wn = W[:, IN_DIM:].T * (1.0 / K)
    b2 = b.reshape(1, OUT_DIM)
    gamma2 = gamma.reshape(1, OUT_DIM)
    beta2 = beta.reshape(1, OUT_DIM)

    grid = (NB + NBO,)
    out = pl.pallas_call(
        _body,
        grid=grid,
        in_specs=[
            pl.BlockSpec((BN, IN_DIM), lambda i: (jnp.minimum(i, NB - 1), 0)),
            pl.BlockSpec((BN, K, IN_DIM), lambda i: (jnp.minimum(i, NB - 1), 0, 0)),
            pl.BlockSpec((IN_DIM, OUT_DIM), lambda i: (0, 0)),
            pl.BlockSpec((IN_DIM, OUT_DIM), lambda i: (0, 0)),
            pl.BlockSpec((1, OUT_DIM), lambda i: (0, 0)),
            pl.BlockSpec((1, OUT_DIM), lambda i: (0, 0)),
            pl.BlockSpec((1, OUT_DIM), lambda i: (0, 0)),
        ],
        out_specs=pl.BlockSpec((BNO, OUT_DIM), lambda i: (jnp.maximum(i - NB, 0), 0)),
        out_shape=jax.ShapeDtypeStruct((N, OUT_DIM), jnp.float32),
        scratch_shapes=[
            pltpu.VMEM((N, OUT_DIM), jnp.float32),
            pltpu.VMEM((2, OUT_DIM), jnp.float32),
        ],
        compiler_params=pltpu.CompilerParams(
            vmem_limit_bytes=100 * 1024 * 1024,
        ),
    )(self_feat, neighbor_feat, ws, wn, b2, gamma2, beta2)
    return out


# speedup vs baseline: 2.4572x; 1.0069x over previous
import functools

import jax
import jax.numpy as jnp
from jax.experimental import pallas as pl
from jax.experimental.pallas import tpu as pltpu

N = 10000
K = 16
IN_DIM = 256
OUT_DIM = 256
BN = 1000
NB = N // BN
BNO = 5000
NBO = N // BNO
EPS = 1e-5


def _body(self_ref, neigh_ref, ws_ref, wn_ref, b_ref, gamma_ref,
          beta_ref, out_ref, y_ref, acc_ref):
    i = pl.program_id(0)

    @pl.when(i == 0)
    def _init():
        acc_ref[...] = jnp.zeros_like(acc_ref)

    @pl.when(i < NB)
    def _compute():
        agg = jnp.sum(neigh_ref[...], axis=1)
        y = (
            jnp.dot(self_ref[...], ws_ref[...], preferred_element_type=jnp.float32)
            + jnp.dot(agg, wn_ref[...], preferred_element_type=jnp.float32)
            + b_ref[...]
        )
        y_ref[pl.ds(i * BN, BN), :] = y
        acc_ref[0:1, :] += jnp.sum(y, axis=0, keepdims=True)
        acc_ref[1:2, :] += jnp.sum(y * y, axis=0, keepdims=True)

    @pl.when(i >= NB)
    def _normalize():
        j = i - NB
        mean = acc_ref[0:1, :] / N
        var = acc_ref[1:2, :] / N - mean * mean
        scale = gamma_ref[...] * jax.lax.rsqrt(var + EPS)
        shift = beta_ref[...] - mean * scale
        y = y_ref[pl.ds(j * BNO, BNO), :]
        out_ref[...] = jnp.maximum(y * scale + shift, 0.0)


@jax.jit
def kernel(self_feat, neighbor_feat, W, b, gamma, beta):
    ws = W[:, :IN_DIM].T
    wn = W[:, IN_DIM:].T * (1.0 / K)
    b2 = b.reshape(1, OUT_DIM)
    gamma2 = gamma.reshape(1, OUT_DIM)
    beta2 = beta.reshape(1, OUT_DIM)

    grid = (NB + NBO,)
    out = pl.pallas_call(
        _body,
        grid=grid,
        in_specs=[
            pl.BlockSpec((BN, IN_DIM), lambda i: (jnp.minimum(i, NB - 1), 0)),
            pl.BlockSpec((BN, K, IN_DIM), lambda i: (jnp.minimum(i, NB - 1), 0, 0)),
            pl.BlockSpec((IN_DIM, OUT_DIM), lambda i: (0, 0)),
            pl.BlockSpec((IN_DIM, OUT_DIM), lambda i: (0, 0)),
            pl.BlockSpec((1, OUT_DIM), lambda i: (0, 0)),
            pl.BlockSpec((1, OUT_DIM), lambda i: (0, 0)),
            pl.BlockSpec((1, OUT_DIM), lambda i: (0, 0)),
        ],
        out_specs=pl.BlockSpec((BNO, OUT_DIM), lambda i: (jnp.maximum(i - NB, 0), 0)),
        out_shape=jax.ShapeDtypeStruct((N, OUT_DIM), jnp.float32),
        scratch_shapes=[
            pltpu.VMEM((N, OUT_DIM), jnp.float32),
            pltpu.VMEM((2, OUT_DIM), jnp.float32),
        ],
        compiler_params=pltpu.CompilerParams(
            vmem_limit_bytes=100 * 1024 * 1024,
        ),
    )(self_feat, neighbor_feat, ws, wn, b2, gamma2, beta2)
    return out
